# Initial kernel scaffold; baseline (speedup 1.0000x reference)
import jax
import jax.numpy as jnp
from jax.experimental import pallas as pl


def kernel(x, edge_index, anchors, Wq, Wk, Wv, W1, b1, W2, b2):
    raise NotImplementedError("write your pallas kernel here")


# speedup vs baseline: 234.8114x; 234.8114x over previous
import functools
import math

import jax
import jax.numpy as jnp
from jax import lax
from jax.experimental import pallas as pl
from jax.experimental.pallas import tpu as pltpu
from jax.experimental.pallas import tpu_sc as plsc

_H = 4


def _make_bfs_kernel(N, E, B):
    mesh = plsc.VectorSubcoreMesh(core_axis_name="c", subcore_axis_name="s")

    @functools.partial(
        pl.kernel,
        mesh=mesh,
        out_type=jax.ShapeDtypeStruct((B, N), jnp.int32),
        compiler_params=pltpu.CompilerParams(needs_layout_passes=False),
        scratch_types=[
            pltpu.VMEM((N,), jnp.int32),
            pltpu.VMEM((E,), jnp.int32),
            pltpu.VMEM((E,), jnp.int32),
            pltpu.VMEM((16,), jnp.int32),
        ],
    )
    def bfs(edge_hbm, anch_hbm, dist_hbm, dist_v, src_v, dst_v, anch_v):
        c = lax.axis_index("c")
        s = lax.axis_index("s")

        @pl.when(s == 0)
        def _():
            b = c
            pltpu.sync_copy(edge_hbm.at[0], src_v)
            pltpu.sync_copy(edge_hbm.at[1], dst_v)
            pltpu.sync_copy(anch_hbm, anch_v)
            anchor = plsc.load_gather(anch_v, [jnp.full((16,), b, jnp.int32)])

            def init_body(i, carry):
                lane = lax.iota(jnp.int32, 16) + i * 16
                dist_v[pl.ds(i * 16, 16)] = jnp.where(lane == anchor, 0, N)
                return carry

            lax.fori_loop(0, N // 16, init_body, 0)

            def sweep(carry):
                t, _ = carry

                def edge_body(e, ch):
                    su = src_v[pl.ds(e * 16, 16)]
                    sv = dst_v[pl.ds(e * 16, 16)]
                    du = plsc.load_gather(dist_v, [su])
                    dv = plsc.load_gather(dist_v, [sv])
                    tv = jnp.full((16,), t, jnp.int32)
                    nv = tv + 1
                    m1 = (du == tv) & (dv == N)
                    plsc.store_scatter(dist_v, [sv], nv, mask=m1)
                    m2 = (dv == tv) & (du == N)
                    plsc.store_scatter(dist_v, [su], nv, mask=m2)
                    return ch | jnp.any(m1 | m2)

                changed = lax.fori_loop(0, E // 16, edge_body, False)
                return t + 1, changed

            def not_done(carry):
                t, changed = carry
                return changed & (t < N)

            lax.while_loop(not_done, sweep, (0, True))
            pltpu.sync_copy(dist_v, dist_hbm.at[b])

    return bfs


def _attn_body(x_ref, dist_ref, wq_ref, wk_ref, wv_ref, w1_ref, b1_ref,
               w2_ref, b2_ref, feat_ref, score_ref, q_ref, kv_ref):
    C = x_ref.shape[1]
    N = x_ref.shape[2]
    Dh = C // _H
    f32 = jnp.float32

    xb = x_ref[0]
    dist = dist_ref[0]
    q_ref[...] = jnp.dot(wq_ref[...], xb, preferred_element_type=f32)
    kv_ref[...] = xb[:, :C]
    feat_ref[0] = xb

    lm = jnp.max(jnp.where(dist < N, dist, -1))
    scale = f32(1.0 / math.sqrt(Dh))

    def level_body(l, carry):
        K = jnp.dot(wk_ref[...], kv_ref[...], preferred_element_type=f32)
        V = jnp.dot(wv_ref[...], kv_ref[...], preferred_element_type=f32)
        mask = dist == l
        for h in range(_H):
            sl = pl.ds(h * Dh, Dh)
            qh = q_ref[sl, :]
            kh = K[h * Dh:(h + 1) * Dh, :]
            vh = V[h * Dh:(h + 1) * Dh, :]
            logits = lax.dot_general(
                qh, kh, (((0,), (0,)), ((), ())),
                preferred_element_type=f32) * scale
            mx = jnp.max(logits, axis=1, keepdims=True)
            ex = jnp.exp(logits - mx)
            a = ex / jnp.sum(ex, axis=1, keepdims=True)
            oh = lax.dot_general(
                vh, a, (((1,), (1,)), ((), ())),
                preferred_element_type=f32)
            updh = oh + x_ref[0, sl, :]
            feat_ref[0, sl, :] = jnp.where(mask, updh, feat_ref[0, sl, :])
            kv_ref[sl, :] = jnp.where(mask[:, :C], updh[:, :C], kv_ref[sl, :])
        return carry

    lax.fori_loop(0, lm + 1, level_body, 0)

    ff = feat_ref[0]
    h1 = jnp.dot(w1_ref[...], ff, preferred_element_type=f32) + b1_ref[...]
    h1 = jnp.maximum(h1, 0.0)
    sc = jnp.dot(w2_ref[...], h1, preferred_element_type=f32) + b2_ref[...]
    score_ref[0] = 1.0 / (1.0 + jnp.exp(-sc))


def _attention_call(x, dist3, Wq, Wk, Wv, W1, b1c, W2, b2c):
    B, C, N = x.shape
    f32 = jnp.float32
    full = lambda shp: pl.BlockSpec(shp, lambda b: (0,) * len(shp))
    feat, score = pl.pallas_call(
        _attn_body,
        grid=(B,),
        in_specs=[
            pl.BlockSpec((1, C, N), lambda b: (b, 0, 0)),
            pl.BlockSpec((1, 1, N), lambda b: (b, 0, 0)),
            full((C, C)), full((C, C)), full((C, C)), full((C, C)),
            full((C, 1)), full((1, C)), full((1, 1)),
        ],
        out_specs=[
            pl.BlockSpec((1, C, N), lambda b: (b, 0, 0)),
            pl.BlockSpec((1, 1, N), lambda b: (b, 0, 0)),
        ],
        out_shape=[
            jax.ShapeDtypeStruct((B, C, N), f32),
            jax.ShapeDtypeStruct((B, 1, N), f32),
        ],
        scratch_shapes=[
            pltpu.VMEM((C, N), f32),
            pltpu.VMEM((C, C), f32),
        ],
    )(x, dist3, Wq, Wk, Wv, W1, b1c, W2, b2c)
    return feat, score


def kernel(x, edge_index, anchors, Wq, Wk, Wv, W1, b1, W2, b2):
    B, C, N = x.shape
    E = edge_index.shape[1]
    anch_pad = jnp.pad(anchors.astype(jnp.int32), (0, 16 - B))
    dist = _make_bfs_kernel(N, E, B)(edge_index, anch_pad)
    dist3 = dist.reshape(B, 1, N)
    feat, score = _attention_call(
        x, dist3, Wq, Wk, Wv, W1,
        b1.reshape(C, 1), W2, b2.reshape(1, 1))
    return feat, score.reshape(B, N, 1)


# speedup vs baseline: 430.2222x; 1.8322x over previous
import functools
import math

import jax
import jax.numpy as jnp
from jax import lax
from jax.experimental import pallas as pl
from jax.experimental.pallas import tpu as pltpu
from jax.experimental.pallas import tpu_sc as plsc

_H = 4


def _make_bfs_kernel(N, E, B):
    mesh = plsc.VectorSubcoreMesh(core_axis_name="c", subcore_axis_name="s")

    @functools.partial(
        pl.kernel,
        mesh=mesh,
        out_type=jax.ShapeDtypeStruct((B, N), jnp.int32),
        compiler_params=pltpu.CompilerParams(needs_layout_passes=False),
        scratch_types=[
            pltpu.VMEM((N,), jnp.int32),
            pltpu.VMEM((E,), jnp.int32),
            pltpu.VMEM((E,), jnp.int32),
            pltpu.VMEM((16,), jnp.int32),
        ],
    )
    def bfs(edge_hbm, anch_hbm, dist_hbm, dist_v, src_v, dst_v, anch_v):
        c = lax.axis_index("c")
        s = lax.axis_index("s")

        @pl.when(s == 0)
        def _():
            b = c
            pltpu.sync_copy(edge_hbm.at[0], src_v)
            pltpu.sync_copy(edge_hbm.at[1], dst_v)
            pltpu.sync_copy(anch_hbm, anch_v)
            anchor = plsc.load_gather(anch_v, [jnp.full((16,), b, jnp.int32)])

            def init_body(i, carry):
                lane = lax.iota(jnp.int32, 16) + i * 16
                dist_v[pl.ds(i * 16, 16)] = jnp.where(lane == anchor, 0, N)
                return carry

            lax.fori_loop(0, N // 16, init_body, 0)

            def sweep(carry):
                t, _, cnt = carry

                @plsc.parallel_loop(0, E // 16, unroll=4)
                def _(e):
                    su = src_v[pl.ds(e * 16, 16)]
                    sv = dst_v[pl.ds(e * 16, 16)]
                    du = plsc.load_gather(dist_v, [su])
                    dv = plsc.load_gather(dist_v, [sv])
                    tv = jnp.full((16,), t, jnp.int32)
                    nv = tv + 1
                    m1 = (du == tv) & (dv == N)
                    plsc.store_scatter(dist_v, [sv], nv, mask=m1)
                    m2 = (dv == tv) & (du == N)
                    plsc.store_scatter(dist_v, [su], nv, mask=m2)

                @plsc.parallel_loop(0, N // 16, unroll=8,
                                    carry=jnp.zeros((16,), jnp.int32))
                def cnts(i, acc):
                    d = dist_v[pl.ds(i * 16, 16)]
                    return acc + jnp.where(d < N, 1, 0)

                return t + 1, cnt, jnp.sum(cnts)

            def not_done(carry):
                t, prev, cnt = carry
                return (cnt != prev) & (t < N)

            lax.while_loop(not_done, sweep, (0, -1, 1))
            pltpu.sync_copy(dist_v, dist_hbm.at[b])

    return bfs


def _attn_body(x_ref, dist_ref, wq_ref, wk_ref, wv_ref, w1_ref, b1_ref,
               w2_ref, b2_ref, feat_ref, score_ref, q_ref, kv_ref):
    C = x_ref.shape[1]
    N = x_ref.shape[2]
    Dh = C // _H
    f32 = jnp.float32

    xb = x_ref[0]
    dist = dist_ref[0]
    q_ref[...] = jnp.dot(wq_ref[...], xb, preferred_element_type=f32)
    kv_ref[...] = xb[:, :C]
    feat_ref[0] = xb

    lm = jnp.max(jnp.where(dist < N, dist, -1))
    scale = f32(1.0 / math.sqrt(Dh))

    def level_body(l, carry):
        K = jnp.dot(wk_ref[...], kv_ref[...], preferred_element_type=f32)
        V = jnp.dot(wv_ref[...], kv_ref[...], preferred_element_type=f32)
        mask = dist == l
        for h in range(_H):
            sl = pl.ds(h * Dh, Dh)
            qh = q_ref[sl, :]
            kh = K[h * Dh:(h + 1) * Dh, :]
            vh = V[h * Dh:(h + 1) * Dh, :]
            logits = lax.dot_general(
                kh, qh, (((0,), (0,)), ((), ())),
                preferred_element_type=f32) * scale
            mx = jnp.max(logits, axis=0, keepdims=True)
            ex = jnp.exp(logits - mx)
            a = ex / jnp.sum(ex, axis=0, keepdims=True)
            oh = lax.dot_general(
                vh, a, (((1,), (0,)), ((), ())),
                preferred_element_type=f32)
            updh = oh + x_ref[0, sl, :]
            feat_ref[0, sl, :] = jnp.where(mask, updh, feat_ref[0, sl, :])
            kv_ref[sl, :] = jnp.where(mask[:, :C], updh[:, :C], kv_ref[sl, :])
        return carry

    lax.fori_loop(0, lm + 1, level_body, 0)

    ff = feat_ref[0]
    h1 = jnp.dot(w1_ref[...], ff, preferred_element_type=f32) + b1_ref[...]
    h1 = jnp.maximum(h1, 0.0)
    sc = jnp.dot(w2_ref[...], h1, preferred_element_type=f32) + b2_ref[...]
    score_ref[0] = 1.0 / (1.0 + jnp.exp(-sc))


def _attention_call(x, dist3, Wq, Wk, Wv, W1, b1c, W2, b2c):
    B, C, N = x.shape
    f32 = jnp.float32
    full = lambda shp: pl.BlockSpec(shp, lambda b: (0,) * len(shp))
    feat, score = pl.pallas_call(
        _attn_body,
        grid=(B,),
        in_specs=[
            pl.BlockSpec((1, C, N), lambda b: (b, 0, 0)),
            pl.BlockSpec((1, 1, N), lambda b: (b, 0, 0)),
            full((C, C)), full((C, C)), full((C, C)), full((C, C)),
            full((C, 1)), full((1, C)), full((1, 1)),
        ],
        out_specs=[
            pl.BlockSpec((1, C, N), lambda b: (b, 0, 0)),
            pl.BlockSpec((1, 1, N), lambda b: (b, 0, 0)),
        ],
        out_shape=[
            jax.ShapeDtypeStruct((B, C, N), f32),
            jax.ShapeDtypeStruct((B, 1, N), f32),
        ],
        scratch_shapes=[
            pltpu.VMEM((C, N), f32),
            pltpu.VMEM((C, C), f32),
        ],
    )(x, dist3, Wq, Wk, Wv, W1, b1c, W2, b2c)
    return feat, score


def kernel(x, edge_index, anchors, Wq, Wk, Wv, W1, b1, W2, b2):
    B, C, N = x.shape
    E = edge_index.shape[1]
    anch_pad = jnp.pad(anchors.astype(jnp.int32), (0, 16 - B))
    dist = _make_bfs_kernel(N, E, B)(edge_index, anch_pad)
    dist3 = dist.reshape(B, 1, N)
    feat, score = _attention_call(
        x, dist3, Wq, Wk, Wv, W1,
        b1.reshape(C, 1), W2, b2.reshape(1, 1))
    return feat, score.reshape(B, N, 1)


# speedup vs baseline: 469.3777x; 1.0910x over previous
import functools
import math

import jax
import jax.numpy as jnp
from jax import lax
from jax.experimental import pallas as pl
from jax.experimental.pallas import tpu as pltpu
from jax.experimental.pallas import tpu_sc as plsc

_H = 4


def _make_bfs_kernel(N, E, B):
    mesh = plsc.VectorSubcoreMesh(core_axis_name="c", subcore_axis_name="s")

    @functools.partial(
        pl.kernel,
        mesh=mesh,
        out_type=jax.ShapeDtypeStruct((B, N), jnp.int32),
        compiler_params=pltpu.CompilerParams(needs_layout_passes=False),
        scratch_types=[
            pltpu.VMEM((N,), jnp.int32),
            pltpu.VMEM((E,), jnp.int32),
            pltpu.VMEM((E,), jnp.int32),
            pltpu.VMEM((16,), jnp.int32),
        ],
    )
    def bfs(edge_hbm, anch_hbm, dist_hbm, dist_v, src_v, dst_v, anch_v):
        c = lax.axis_index("c")
        s = lax.axis_index("s")

        @pl.when(s == 0)
        def _():
            b = c
            pltpu.sync_copy(edge_hbm.at[0], src_v)
            pltpu.sync_copy(edge_hbm.at[1], dst_v)
            pltpu.sync_copy(anch_hbm, anch_v)
            anchor = plsc.load_gather(anch_v, [jnp.full((16,), b, jnp.int32)])

            def init_body(i, carry):
                lane = lax.iota(jnp.int32, 16) + i * 16
                dist_v[pl.ds(i * 16, 16)] = jnp.where(lane == anchor, 0, N)
                return carry

            lax.fori_loop(0, N // 16, init_body, 0)

            def sweep(carry):
                t, _, cnt = carry

                @plsc.parallel_loop(0, E // 16, unroll=8)
                def _(e):
                    su = src_v[pl.ds(e * 16, 16)]
                    sv = dst_v[pl.ds(e * 16, 16)]
                    du = plsc.load_gather(dist_v, [su])
                    dv = plsc.load_gather(dist_v, [sv])
                    tv = jnp.full((16,), t, jnp.int32)
                    nv = tv + 1
                    m1 = (du == tv) & (dv == N)
                    plsc.store_scatter(dist_v, [sv], nv, mask=m1)
                    m2 = (dv == tv) & (du == N)
                    plsc.store_scatter(dist_v, [su], nv, mask=m2)

                @plsc.parallel_loop(0, N // 16, unroll=8,
                                    carry=jnp.zeros((16,), jnp.int32))
                def cnts(i, acc):
                    d = dist_v[pl.ds(i * 16, 16)]
                    return acc + jnp.where(d < N, 1, 0)

                return t + 1, cnt, jnp.sum(cnts)

            def not_done(carry):
                t, prev, cnt = carry
                return (cnt != prev) & (t < N)

            lax.while_loop(not_done, sweep, (0, -1, 1))
            pltpu.sync_copy(dist_v, dist_hbm.at[b])

    return bfs


def _attn_body(x_ref, dist_ref, wq_ref, wk_ref, wv_ref, w1_ref, b1_ref,
               w2_ref, b2_ref, feat_ref, score_ref, q_ref, kv_ref):
    C = x_ref.shape[1]
    N = x_ref.shape[2]
    Dh = C // _H
    f32 = jnp.float32

    xb = x_ref[0]
    dist = dist_ref[0]
    qscale = f32((1.0 / math.sqrt(Dh)) * math.log2(math.e))
    q_ref[...] = jnp.dot(wq_ref[...], xb, preferred_element_type=f32) * qscale
    kv_ref[...] = xb[:, :C]
    feat_ref[0] = xb

    lm = jnp.max(jnp.where(dist < N, dist, -1))

    def level_body(l, carry):
        K = jnp.dot(wk_ref[...], kv_ref[...], preferred_element_type=f32)
        V = jnp.dot(wv_ref[...], kv_ref[...], preferred_element_type=f32)
        mask = dist == l
        for h in range(_H):
            sl = pl.ds(h * Dh, Dh)
            qh = q_ref[sl, :]
            kh = K[h * Dh:(h + 1) * Dh, :]
            vh = V[h * Dh:(h + 1) * Dh, :]
            logits = lax.dot_general(
                kh, qh, (((0,), (0,)), ((), ())),
                preferred_element_type=f32)
            mx = jnp.max(logits, axis=0, keepdims=True)
            ex = jnp.exp2(logits - mx)
            oh = lax.dot_general(
                vh, ex, (((1,), (0,)), ((), ())),
                preferred_element_type=f32)
            oh = oh * (1.0 / jnp.sum(ex, axis=0, keepdims=True))
            updh = oh + x_ref[0, sl, :]
            feat_ref[0, sl, :] = jnp.where(mask, updh, feat_ref[0, sl, :])
            kv_ref[sl, :] = jnp.where(mask[:, :C], updh[:, :C], kv_ref[sl, :])
        return carry

    lax.fori_loop(0, lm + 1, level_body, 0)

    ff = feat_ref[0]
    h1 = jnp.dot(w1_ref[...], ff, preferred_element_type=f32) + b1_ref[...]
    h1 = jnp.maximum(h1, 0.0)
    sc = jnp.dot(w2_ref[...], h1, preferred_element_type=f32) + b2_ref[...]
    score_ref[0] = 1.0 / (1.0 + jnp.exp(-sc))


def _attention_call(x, dist3, Wq, Wk, Wv, W1, b1c, W2, b2c):
    B, C, N = x.shape
    f32 = jnp.float32
    full = lambda shp: pl.BlockSpec(shp, lambda b: (0,) * len(shp))
    feat, score = pl.pallas_call(
        _attn_body,
        grid=(B,),
        in_specs=[
            pl.BlockSpec((1, C, N), lambda b: (b, 0, 0)),
            pl.BlockSpec((1, 1, N), lambda b: (b, 0, 0)),
            full((C, C)), full((C, C)), full((C, C)), full((C, C)),
            full((C, 1)), full((1, C)), full((1, 1)),
        ],
        out_specs=[
            pl.BlockSpec((1, C, N), lambda b: (b, 0, 0)),
            pl.BlockSpec((1, 1, N), lambda b: (b, 0, 0)),
        ],
        out_shape=[
            jax.ShapeDtypeStruct((B, C, N), f32),
            jax.ShapeDtypeStruct((B, 1, N), f32),
        ],
        scratch_shapes=[
            pltpu.VMEM((C, N), f32),
            pltpu.VMEM((C, C), f32),
        ],
    )(x, dist3, Wq, Wk, Wv, W1, b1c, W2, b2c)
    return feat, score


def kernel(x, edge_index, anchors, Wq, Wk, Wv, W1, b1, W2, b2):
    B, C, N = x.shape
    E = edge_index.shape[1]
    anch_pad = jnp.pad(anchors.astype(jnp.int32), (0, 16 - B))
    dist = _make_bfs_kernel(N, E, B)(edge_index, anch_pad)
    dist3 = dist.reshape(B, 1, N)
    feat, score = _attention_call(
        x, dist3, Wq, Wk, Wv, W1,
        b1.reshape(C, 1), W2, b2.reshape(1, 1))
    return feat, score.reshape(B, N, 1)


# speedup vs baseline: 545.6309x; 1.1625x over previous
import functools
import math

import jax
import jax.numpy as jnp
from jax import lax
from jax.experimental import pallas as pl
from jax.experimental.pallas import tpu as pltpu
from jax.experimental.pallas import tpu_sc as plsc

_H = 4


def _make_bfs_kernel(N, E, B):
    mesh = plsc.VectorSubcoreMesh(core_axis_name="c", subcore_axis_name="s")

    @functools.partial(
        pl.kernel,
        mesh=mesh,
        out_type=jax.ShapeDtypeStruct((B, N), jnp.int32),
        compiler_params=pltpu.CompilerParams(needs_layout_passes=False),
        scratch_types=[
            pltpu.VMEM((N,), jnp.int32),
            pltpu.VMEM((E,), jnp.int32),
            pltpu.VMEM((E,), jnp.int32),
            pltpu.VMEM((16,), jnp.int32),
        ],
    )
    def bfs(edge_hbm, anch_hbm, dist_hbm, dist_v, src_v, dst_v, anch_v):
        c = lax.axis_index("c")
        s = lax.axis_index("s")

        @pl.when(s == 0)
        def _():
            b = c
            pltpu.sync_copy(edge_hbm.at[0], src_v)
            pltpu.sync_copy(edge_hbm.at[1], dst_v)
            pltpu.sync_copy(anch_hbm, anch_v)
            anchor = plsc.load_gather(anch_v, [jnp.full((16,), b, jnp.int32)])

            def init_body(i, carry):
                lane = lax.iota(jnp.int32, 16) + i * 16
                dist_v[pl.ds(i * 16, 16)] = jnp.where(lane == anchor, 0, N)
                return carry

            lax.fori_loop(0, N // 16, init_body, 0)

            def sweep(carry):
                t, _, cnt = carry

                @plsc.parallel_loop(0, E // 16, unroll=8)
                def _(e):
                    su = src_v[pl.ds(e * 16, 16)]
                    sv = dst_v[pl.ds(e * 16, 16)]
                    du = plsc.load_gather(dist_v, [su])
                    dv = plsc.load_gather(dist_v, [sv])
                    tv = jnp.full((16,), t, jnp.int32)
                    nv = tv + 1
                    m1 = (du == tv) & (dv == N)
                    plsc.store_scatter(dist_v, [sv], nv, mask=m1)
                    m2 = (dv == tv) & (du == N)
                    plsc.store_scatter(dist_v, [su], nv, mask=m2)

                @plsc.parallel_loop(0, N // 16, unroll=8,
                                    carry=jnp.zeros((16,), jnp.int32))
                def cnts(i, acc):
                    d = dist_v[pl.ds(i * 16, 16)]
                    return acc + jnp.where(d < N, 1, 0)

                return t + 1, cnt, jnp.sum(cnts)

            def not_done(carry):
                t, prev, cnt = carry
                return (cnt != prev) & (t < N)

            lax.while_loop(not_done, sweep, (0, -1, 1))
            pltpu.sync_copy(dist_v, dist_hbm.at[b])

    return bfs


def _attn_body(x_ref, dist_ref, distc_ref, wq_ref, wk_ref, wv_ref, w1_ref,
               b1_ref, w2_ref, b2_ref, feat_ref, score_ref, q_ref, kv_ref):
    C = x_ref.shape[1]
    N = x_ref.shape[2]
    Dh = C // _H
    f32 = jnp.float32

    xb = x_ref[0]
    dist = dist_ref[0]
    distc = distc_ref[0]
    qscale = f32((1.0 / math.sqrt(Dh)) * math.log2(math.e))
    q_ref[...] = jnp.dot(wq_ref[...], xb, preferred_element_type=f32) * qscale
    kv_ref[...] = xb[:, :C]

    dist128 = dist[:, :C]
    lm128 = jnp.max(jnp.where(dist128 < N, dist128, -1))

    def mini_body(l, carry):
        K = jnp.dot(wk_ref[...], kv_ref[...], preferred_element_type=f32)
        V = jnp.dot(wv_ref[...], kv_ref[...], preferred_element_type=f32)
        m128 = dist128 == l
        for h in range(_H):
            rs = slice(h * Dh, (h + 1) * Dh)
            qh = q_ref[rs, :C]
            kh = K[rs, :]
            vh = V[rs, :]
            logits = lax.dot_general(
                kh, qh, (((0,), (0,)), ((), ())),
                preferred_element_type=f32)
            mx = jnp.max(logits, axis=0, keepdims=True)
            ex = jnp.exp2(logits - mx)
            oh = lax.dot_general(
                vh, ex, (((1,), (0,)), ((), ())),
                preferred_element_type=f32)
            oh = oh * (1.0 / jnp.sum(ex, axis=0, keepdims=True))
            updh = oh + x_ref[0, rs, :C]
            kv_ref[rs, :] = jnp.where(m128, updh, kv_ref[rs, :])
        return carry

    lax.fori_loop(0, lm128 + 1, mini_body, 0)

    kv0 = xb[:, :C]
    K0 = jnp.dot(wk_ref[...], kv0, preferred_element_type=f32)
    V0 = jnp.dot(wv_ref[...], kv0, preferred_element_type=f32)
    K1 = jnp.dot(wk_ref[...], kv_ref[...], preferred_element_type=f32)
    V1 = jnp.dot(wv_ref[...], kv_ref[...], preferred_element_type=f32)
    sel = distc < dist
    live = dist < N
    for h in range(_H):
        rs = slice(h * Dh, (h + 1) * Dh)
        qh = q_ref[rs, :]
        l0 = lax.dot_general(K0[rs, :], qh, (((0,), (0,)), ((), ())),
                             preferred_element_type=f32)
        l1 = lax.dot_general(K1[rs, :], qh, (((0,), (0,)), ((), ())),
                             preferred_element_type=f32)
        logits = jnp.where(sel, l1, l0)
        mx = jnp.max(logits, axis=0, keepdims=True)
        ex = jnp.exp2(logits - mx)
        e1 = jnp.where(sel, ex, 0.0)
        e0 = ex - e1
        oh = (lax.dot_general(V1[rs, :], e1, (((1,), (0,)), ((), ())),
                              preferred_element_type=f32)
              + lax.dot_general(V0[rs, :], e0, (((1,), (0,)), ((), ())),
                                preferred_element_type=f32))
        oh = oh * (1.0 / jnp.sum(ex, axis=0, keepdims=True))
        xh = x_ref[0, rs, :]
        feat_ref[0, rs, :] = jnp.where(live, oh + xh, xh)

    ff = feat_ref[0]
    h1 = jnp.dot(w1_ref[...], ff, preferred_element_type=f32) + b1_ref[...]
    h1 = jnp.maximum(h1, 0.0)
    sc = jnp.dot(w2_ref[...], h1, preferred_element_type=f32) + b2_ref[...]
    score_ref[0] = 1.0 / (1.0 + jnp.exp(-sc))


def _attention_call(x, dist3, distc3, Wq, Wk, Wv, W1, b1c, W2, b2c):
    B, C, N = x.shape
    f32 = jnp.float32
    full = lambda shp: pl.BlockSpec(shp, lambda b: (0,) * len(shp))
    feat, score = pl.pallas_call(
        _attn_body,
        grid=(B,),
        in_specs=[
            pl.BlockSpec((1, C, N), lambda b: (b, 0, 0)),
            pl.BlockSpec((1, 1, N), lambda b: (b, 0, 0)),
            pl.BlockSpec((1, C, 1), lambda b: (b, 0, 0)),
            full((C, C)), full((C, C)), full((C, C)), full((C, C)),
            full((C, 1)), full((1, C)), full((1, 1)),
        ],
        out_specs=[
            pl.BlockSpec((1, C, N), lambda b: (b, 0, 0)),
            pl.BlockSpec((1, 1, N), lambda b: (b, 0, 0)),
        ],
        out_shape=[
            jax.ShapeDtypeStruct((B, C, N), f32),
            jax.ShapeDtypeStruct((B, 1, N), f32),
        ],
        scratch_shapes=[
            pltpu.VMEM((C, N), f32),
            pltpu.VMEM((C, C), f32),
        ],
    )(x, dist3, distc3, Wq, Wk, Wv, W1, b1c, W2, b2c)
    return feat, score


def kernel(x, edge_index, anchors, Wq, Wk, Wv, W1, b1, W2, b2):
    B, C, N = x.shape
    E = edge_index.shape[1]
    anch_pad = jnp.pad(anchors.astype(jnp.int32), (0, 16 - B))
    dist = _make_bfs_kernel(N, E, B)(edge_index, anch_pad)
    dist3 = dist.reshape(B, 1, N)
    distc3 = dist[:, :C].reshape(B, C, 1)
    feat, score = _attention_call(
        x, dist3, distc3, Wq, Wk, Wv, W1,
        b1.reshape(C, 1), W2, b2.reshape(1, 1))
    return feat, score.reshape(B, N, 1)


# speedup vs baseline: 560.5947x; 1.0274x over previous
import functools
import math

import jax
import jax.numpy as jnp
from jax import lax
from jax.experimental import pallas as pl
from jax.experimental.pallas import tpu as pltpu
from jax.experimental.pallas import tpu_sc as plsc

_H = 4


def _make_bfs_kernel(N, E, B):
    mesh = plsc.VectorSubcoreMesh(core_axis_name="c", subcore_axis_name="s")

    @functools.partial(
        pl.kernel,
        mesh=mesh,
        out_type=jax.ShapeDtypeStruct((B, N), jnp.int32),
        compiler_params=pltpu.CompilerParams(needs_layout_passes=False),
        scratch_types=[
            pltpu.VMEM((N,), jnp.int32),
            pltpu.VMEM((E,), jnp.int32),
            pltpu.VMEM((E,), jnp.int32),
            pltpu.VMEM((16,), jnp.int32),
        ],
    )
    def bfs(edge_hbm, anch_hbm, dist_hbm, dist_v, src_v, dst_v, anch_v):
        c = lax.axis_index("c")
        s = lax.axis_index("s")

        @pl.when(s == 0)
        def _():
            b = c
            pltpu.sync_copy(edge_hbm.at[0], src_v)
            pltpu.sync_copy(edge_hbm.at[1], dst_v)
            pltpu.sync_copy(anch_hbm, anch_v)
            anchor = plsc.load_gather(anch_v, [jnp.full((16,), b, jnp.int32)])

            def init_body(i, carry):
                lane = lax.iota(jnp.int32, 16) + i * 16
                dist_v[pl.ds(i * 16, 16)] = jnp.where(lane == anchor, 0, N)
                return carry

            lax.fori_loop(0, N // 16, init_body, 0)

            def sweep(carry):
                t, _, cnt = carry

                @plsc.parallel_loop(0, E // 16, unroll=8)
                def _(e):
                    su = src_v[pl.ds(e * 16, 16)]
                    sv = dst_v[pl.ds(e * 16, 16)]
                    du = plsc.load_gather(dist_v, [su])
                    dv = plsc.load_gather(dist_v, [sv])
                    tv = jnp.full((16,), t, jnp.int32)
                    nv = tv + 1
                    m1 = (du == tv) & (dv == N)
                    plsc.store_scatter(dist_v, [sv], nv, mask=m1)
                    m2 = (dv == tv) & (du == N)
                    plsc.store_scatter(dist_v, [su], nv, mask=m2)

                @plsc.parallel_loop(0, N // 16, unroll=8,
                                    carry=jnp.zeros((16,), jnp.int32))
                def cnts(i, acc):
                    d = dist_v[pl.ds(i * 16, 16)]
                    return acc + jnp.where(d < N, 1, 0)

                return t + 1, cnt, jnp.sum(cnts)

            def not_done(carry):
                t, prev, cnt = carry
                return (cnt != prev) & (t < N)

            lax.while_loop(not_done, sweep, (0, -1, 1))
            pltpu.sync_copy(dist_v, dist_hbm.at[b])

    return bfs


def _attn_body(x_ref, dist_ref, distc_ref, wq_ref, wk_ref, wv_ref, w1_ref,
               b1_ref, w2_ref, b2_ref, feat_ref, score_ref, q_ref, kv_ref):
    B = x_ref.shape[0]
    C = x_ref.shape[1]
    N = x_ref.shape[2]
    Dh = C // _H
    f32 = jnp.float32

    qscale = f32((1.0 / math.sqrt(Dh)) * math.log2(math.e))
    for b in range(B):
        q_ref[b] = jnp.dot(wq_ref[...], x_ref[b],
                           preferred_element_type=f32) * qscale
        kv_ref[b] = x_ref[b, :, :C]

    distc_all = distc_ref[...]
    lm128 = jnp.max(jnp.where(distc_all < N, distc_all, -1))

    def mini_body(l, carry):
        for b in range(B):
            K = jnp.dot(wk_ref[...], kv_ref[b], preferred_element_type=f32)
            V = jnp.dot(wv_ref[...], kv_ref[b], preferred_element_type=f32)
            m128 = dist_ref[b, :, :C] == l
            for h in range(_H):
                rs = slice(h * Dh, (h + 1) * Dh)
                qh = q_ref[b, rs, :C]
                kh = K[rs, :]
                vh = V[rs, :]
                logits = lax.dot_general(
                    kh, qh, (((0,), (0,)), ((), ())),
                    preferred_element_type=f32)
                mx = jnp.max(logits, axis=0, keepdims=True)
                ex = jnp.exp2(logits - mx)
                oh = lax.dot_general(
                    vh, ex, (((1,), (0,)), ((), ())),
                    preferred_element_type=f32)
                oh = oh * (1.0 / jnp.sum(ex, axis=0, keepdims=True))
                updh = oh + x_ref[b, rs, :C]
                kv_ref[b, rs, :] = jnp.where(m128, updh, kv_ref[b, rs, :])
        return carry

    lax.fori_loop(0, lm128 + 1, mini_body, 0)

    for b in range(B):
        kv0 = x_ref[b, :, :C]
        K0 = jnp.dot(wk_ref[...], kv0, preferred_element_type=f32)
        V0 = jnp.dot(wv_ref[...], kv0, preferred_element_type=f32)
        K1 = jnp.dot(wk_ref[...], kv_ref[b], preferred_element_type=f32)
        V1 = jnp.dot(wv_ref[...], kv_ref[b], preferred_element_type=f32)
        dist = dist_ref[b]
        sel = distc_ref[b] < dist
        live = dist < N
        for h in range(_H):
            rs = slice(h * Dh, (h + 1) * Dh)
            qh = q_ref[b, rs, :]
            kcat = jnp.concatenate([K0[rs, :], K1[rs, :]], axis=1)
            lcat = lax.dot_general(kcat, qh, (((0,), (0,)), ((), ())),
                                   preferred_element_type=f32)
            logits = jnp.where(sel, lcat[C:, :], lcat[:C, :])
            mx = jnp.max(logits, axis=0, keepdims=True)
            ex = jnp.exp2(logits - mx)
            e1 = jnp.where(sel, ex, 0.0)
            e0 = ex - e1
            oh = (lax.dot_general(V1[rs, :], e1, (((1,), (0,)), ((), ())),
                                  preferred_element_type=f32)
                  + lax.dot_general(V0[rs, :], e0, (((1,), (0,)), ((), ())),
                                    preferred_element_type=f32))
            oh = oh * (1.0 / jnp.sum(ex, axis=0, keepdims=True))
            xh = x_ref[b, rs, :]
            feat_ref[b, rs, :] = jnp.where(live, oh + xh, xh)

        ff = feat_ref[b]
        h1 = jnp.dot(w1_ref[...], ff, preferred_element_type=f32) + b1_ref[...]
        h1 = jnp.maximum(h1, 0.0)
        sc = jnp.dot(w2_ref[...], h1, preferred_element_type=f32) + b2_ref[...]
        score_ref[b] = 1.0 / (1.0 + jnp.exp(-sc))


def _attention_call(x, dist3, distc3, Wq, Wk, Wv, W1, b1c, W2, b2c):
    B, C, N = x.shape
    f32 = jnp.float32
    feat, score = pl.pallas_call(
        _attn_body,
        out_shape=[
            jax.ShapeDtypeStruct((B, C, N), f32),
            jax.ShapeDtypeStruct((B, 1, N), f32),
        ],
        scratch_shapes=[
            pltpu.VMEM((B, C, N), f32),
            pltpu.VMEM((B, C, C), f32),
        ],
    )(x, dist3, distc3, Wq, Wk, Wv, W1, b1c, W2, b2c)
    return feat, score


def kernel(x, edge_index, anchors, Wq, Wk, Wv, W1, b1, W2, b2):
    B, C, N = x.shape
    E = edge_index.shape[1]
    anch_pad = jnp.pad(anchors.astype(jnp.int32), (0, 16 - B))
    dist = _make_bfs_kernel(N, E, B)(edge_index, anch_pad)
    dist3 = dist.reshape(B, 1, N)
    distc3 = dist[:, :C].reshape(B, C, 1)
    feat, score = _attention_call(
        x, dist3, distc3, Wq, Wk, Wv, W1,
        b1.reshape(C, 1), W2, b2.reshape(1, 1))
    return feat, score.reshape(B, N, 1)
